# baseline (speedup 1.0000x reference)
import functools

import jax
import jax.numpy as jnp
from jax import lax
from jax.experimental import pallas as pl
from jax.experimental.pallas import tpu as pltpu
from jax.experimental.pallas import tpu_sc as plsc

B = 10000
D = 256
DP = D // 2
K = 16
EMB = 256

NC = 2
NS = 16
NW = NC * NS
BPW = 320
BP = NW * BPW

BLKN = 8
BLKR = BLKN * K
NBLK = BPW // BLKN
NRING = 2
NIT = NBLK // NRING

SCH = 10
SC_C = 32
PGRP = DP // 16
TPAD = 10112
TROWS = TPAD // NS

_sc_mesh = plsc.VectorSubcoreMesh(core_axis_name="c", subcore_axis_name="s")
_HI = -65536


@functools.partial(
    pl.kernel,
    out_type=[
        jax.ShapeDtypeStruct((BP, DP), jnp.int32),
        jax.ShapeDtypeStruct((BP, DP), jnp.int32),
    ],
    mesh=_sc_mesh,
    scratch_types=[
        pltpu.VMEM((SCH, SC_C), jnp.int32),
        pltpu.VMEM((NBLK, BLKR), jnp.int32),
        pltpu.VMEM((SC_C, DP), jnp.int32),
        pltpu.VMEM((SC_C, DP), jnp.int32),
        pltpu.VMEM((BLKR, DP), jnp.int32),
        pltpu.VMEM((BLKR, DP), jnp.int32),
        pltpu.VMEM((BLKN, DP), jnp.int32),
        pltpu.VMEM((BLKN, DP), jnp.int32),
        pltpu.SemaphoreType.DMA,
        pltpu.SemaphoreType.DMA,
        pltpu.SemaphoreType.DMA,
        pltpu.SemaphoreType.DMA,
        pltpu.SemaphoreType.DMA,
        pltpu.SemaphoreType.DMA,
        pltpu.SemaphoreType.DMA,
        pltpu.SemaphoreType.DMA,
        pltpu.VMEM_SHARED((TPAD, DP), jnp.int32),
        pltpu.SemaphoreType.DMA,
    ],
)
def _sc_gather(nodes_hbm, neigh_hbm, tpack_hbm, self_out, nsum_out,
               nodes_v, neigh_v, svp0, svp1, nb0, nb1,
               osum0, osum1,
               sem_n0, sem_n1, sem_o0, sem_o1,
               sem_si0, sem_si1, sem_so0, sem_so1, tsh, sem_t):
    cid = lax.axis_index("c")
    sid = lax.axis_index("s")
    w = sid * NC + cid
    base = w * BPW

    pltpu.sync_copy(nodes_hbm.at[w], nodes_v)
    pltpu.sync_copy(neigh_hbm.at[w], neigh_v)

    h_t = pltpu.async_copy(
        tpack_hbm.at[pl.ds(sid * TROWS, TROWS)],
        tsh.at[pl.ds(sid * TROWS, TROWS)], sem_t)

    nbs = (nb0, nb1)
    osums = (osum0, osum1)
    sem_ns = (sem_n0, sem_n1)
    sem_os = (sem_o0, sem_o1)

    shift16 = jnp.full((16,), 16, jnp.int32)
    himask = jnp.full((16,), _HI, jnp.int32)
    rnd = jnp.full((16,), 32768, jnp.int32)

    def lo_f32(v):
        return lax.bitcast_convert_type(lax.shift_left(v, shift16), jnp.float32)

    def hi_f32(v):
        return lax.bitcast_convert_type(lax.bitwise_and(v, himask), jnp.float32)

    def rn_pack(acc_a, acc_b):
        ia = lax.bitcast_convert_type(acc_a, jnp.int32) + rnd
        ib = lax.bitcast_convert_type(acc_b, jnp.int32) + rnd
        return lax.bitwise_or(lax.shift_right_logical(ia, shift16),
                              lax.bitwise_and(ib, himask))

    def accum_block(buf, osum_b):
        @plsc.parallel_loop(0, BLKN)
        def _(r):
            row = r * K
            for g in range(PGRP):
                sl = pl.ds(g * 16, 16)
                acc_a = jnp.zeros((16,), jnp.float32)
                acc_b = jnp.zeros((16,), jnp.float32)
                for j in range(K):
                    v = buf[row + j, sl]
                    acc_a = acc_a + lo_f32(v)
                    acc_b = acc_b + hi_f32(v)
                osum_b[r, sl] = rn_pack(acc_a, acc_b)

    h_t.wait()
    plsc.subcore_barrier()

    for q in range(NRING):
        pltpu.async_copy(tsh.at[neigh_v.at[q]], nbs[q], sem_ns[q])

    svp = (svp0, svp1)
    sem_si = (sem_si0, sem_si1)
    sem_so = (sem_so0, sem_so1)
    h_in = [pltpu.async_copy(tsh.at[nodes_v.at[0]], svp0, sem_si0),
            pltpu.async_copy(tsh.at[nodes_v.at[1]], svp1, sem_si1)]
    for c in range(SCH):
        bsl = c % 2
        h_in[bsl].wait()
        ho = pltpu.async_copy(
            svp[bsl], self_out.at[pl.ds(base + c * SC_C, SC_C)], sem_so[bsl])
        if c + 2 < SCH:
            ho.wait()
            h_in[bsl] = pltpu.async_copy(
                tsh.at[nodes_v.at[c + 2]], svp[bsl], sem_si[bsl])

    def loop_body(i, _):
        for q in range(NRING):
            blk = i * NRING + q
            row0 = base + i * (NRING * BLKN) + q * BLKN
            pltpu.make_async_copy(
                tsh.at[neigh_v.at[blk]], nbs[q], sem_ns[q]).wait()

            @pl.when(i > 0)
            def _():
                pltpu.make_async_copy(
                    osums[q], nsum_out.at[pl.ds(row0 - NRING * BLKN, BLKN)],
                    sem_os[q]).wait()

            accum_block(nbs[q], osums[q])

            @pl.when(i < NIT - 1)
            def _():
                pltpu.async_copy(
                    tsh.at[neigh_v.at[blk + NRING]], nbs[q], sem_ns[q])

            pltpu.async_copy(
                osums[q], nsum_out.at[pl.ds(row0, BLKN)], sem_os[q])
        return 0

    lax.fori_loop(0, NIT, loop_body, 0)

    for q in range(NRING):
        lastq = base + (NIT - 1) * NRING * BLKN + q * BLKN
        pltpu.make_async_copy(
            osums[q], nsum_out.at[pl.ds(lastq, BLKN)], sem_os[q]).wait()
    pltpu.make_async_copy(
        svp0, self_out.at[pl.ds(base + (SCH - 2) * SC_C, SC_C)], sem_so0).wait()
    pltpu.make_async_copy(
        svp1, self_out.at[pl.ds(base + (SCH - 1) * SC_C, SC_C)], sem_so1).wait()


def _mm_body(x1_ref, x2_ref, w1a_ref, w1b_ref, w2a_ref, w2b_ref, b_ref, o_ref):
    x1 = x1_ref[...]
    x2 = x2_ref[...]
    a1 = jax.lax.bitcast_convert_type(
        jax.lax.shift_left(x1, jnp.int32(16)), jnp.float32)
    b1 = jax.lax.bitcast_convert_type(
        jax.lax.bitwise_and(x1, jnp.int32(_HI)), jnp.float32)
    a2 = jax.lax.bitcast_convert_type(
        jax.lax.shift_left(x2, jnp.int32(16)), jnp.float32)
    b2 = jax.lax.bitcast_convert_type(
        jax.lax.bitwise_and(x2, jnp.int32(_HI)), jnp.float32)
    acc = jnp.dot(a1, w1a_ref[...], preferred_element_type=jnp.float32)
    acc = acc + jnp.dot(b1, w1b_ref[...], preferred_element_type=jnp.float32)
    acc = acc + jnp.dot(a2, w2a_ref[...], preferred_element_type=jnp.float32)
    acc = acc + jnp.dot(b2, w2b_ref[...], preferred_element_type=jnp.float32)
    o_ref[...] = jnp.maximum(acc + b_ref[...], 0.0)


_BM = 1000


def _dense(x1, x2, w1a, w1b, w2a, w2b, b2d):
    return pl.pallas_call(
        _mm_body,
        grid=(B // _BM,),
        in_specs=[
            pl.BlockSpec((_BM, DP), lambda i: (i, 0)),
            pl.BlockSpec((_BM, DP), lambda i: (i, 0)),
            pl.BlockSpec((DP, EMB), lambda i: (0, 0)),
            pl.BlockSpec((DP, EMB), lambda i: (0, 0)),
            pl.BlockSpec((DP, EMB), lambda i: (0, 0)),
            pl.BlockSpec((DP, EMB), lambda i: (0, 0)),
            pl.BlockSpec((1, EMB), lambda i: (0, 0)),
        ],
        out_specs=pl.BlockSpec((_BM, EMB), lambda i: (i, 0)),
        out_shape=jax.ShapeDtypeStruct((B, EMB), jnp.float32),
    )(x1, x2, w1a, w1b, w2a, w2b, b2d)


def kernel(table, nodes, neigh_idx, W, b):
    nodes_i = nodes.astype(jnp.int32)
    neigh_i = neigh_idx.astype(jnp.int32)
    pad = BP - B
    nodes_p = jnp.concatenate([nodes_i, jnp.zeros((pad,), jnp.int32)])
    neigh_p = jnp.concatenate([neigh_i, jnp.zeros((pad, K), jnp.int32)])
    nodes_r = nodes_p.reshape(NW, SCH, SC_C)
    neigh_r = neigh_p.reshape(NW, NBLK, BLKR)

    tb = table.astype(jnp.bfloat16)
    tpack = jax.lax.bitcast_convert_type(tb.reshape(B, DP, 2), jnp.int32)
    tpack = jnp.concatenate(
        [tpack, jnp.zeros((TPAD - B, DP), jnp.int32)])

    selfp, nsump = _sc_gather(nodes_r, neigh_r, tpack)

    p_lo = 2 * jnp.arange(DP)
    p_hi = 2 * jnp.arange(DP) + 1
    w1 = W[:D]
    w2 = W[D:] * (1.0 / K)
    out = _dense(selfp, nsump,
                 w1[p_lo], w1[p_hi], w2[p_lo], w2[p_hi], b.reshape(1, EMB))
    return out


# speedup vs baseline: 1.7385x; 1.7385x over previous
import functools

import jax
import jax.numpy as jnp
from jax import lax
from jax.experimental import pallas as pl
from jax.experimental.pallas import tpu as pltpu
from jax.experimental.pallas import tpu_sc as plsc

B = 10000
D = 256
DP = D // 2
K = 16
EMB = 256

NC = 2
NS = 16
NW = NC * NS
BPW = 320
BP = NW * BPW

BLKN = 8
BLKR = BLKN * K
NBLK = BPW // BLKN
NRING = 2
NIT = NBLK // NRING

SCH = 10
SC_C = 32
PGRP = DP // 16
TPAD = 10112
TROWS = TPAD // NS

_sc_mesh = plsc.VectorSubcoreMesh(core_axis_name="c", subcore_axis_name="s")
_HI = -65536


@functools.partial(
    pl.kernel,
    out_type=[
        jax.ShapeDtypeStruct((BP, DP), jnp.int32),
        jax.ShapeDtypeStruct((BP, DP), jnp.int32),
    ],
    mesh=_sc_mesh,
    scratch_types=[
        pltpu.VMEM((SCH, SC_C), jnp.int32),
        pltpu.VMEM((NBLK, BLKR), jnp.int32),
        pltpu.VMEM((SC_C, DP), jnp.int32),
        pltpu.VMEM((SC_C, DP), jnp.int32),
        pltpu.VMEM((BLKR, DP), jnp.int32),
        pltpu.VMEM((BLKR, DP), jnp.int32),
        pltpu.VMEM((BLKN, DP), jnp.int32),
        pltpu.VMEM((BLKN, DP), jnp.int32),
        pltpu.SemaphoreType.DMA,
        pltpu.SemaphoreType.DMA,
        pltpu.SemaphoreType.DMA,
        pltpu.SemaphoreType.DMA,
        pltpu.SemaphoreType.DMA,
        pltpu.SemaphoreType.DMA,
        pltpu.SemaphoreType.DMA,
        pltpu.SemaphoreType.DMA,
        pltpu.VMEM_SHARED((TPAD, DP), jnp.int32),
        pltpu.SemaphoreType.DMA,
    ],
)
def _sc_gather(nodes_hbm, neigh_hbm, tpack_hbm, self_out, nsum_out,
               nodes_v, neigh_v, svp0, svp1, nb0, nb1,
               osum0, osum1,
               sem_n0, sem_n1, sem_o0, sem_o1,
               sem_si0, sem_si1, sem_so0, sem_so1, tsh, sem_t):
    cid = lax.axis_index("c")
    sid = lax.axis_index("s")
    w = sid * NC + cid
    base = w * BPW

    pltpu.sync_copy(nodes_hbm.at[w], nodes_v)
    pltpu.sync_copy(neigh_hbm.at[w], neigh_v)

    h_t = pltpu.async_copy(
        tpack_hbm.at[pl.ds(sid * TROWS, TROWS)],
        tsh.at[pl.ds(sid * TROWS, TROWS)], sem_t)

    nbs = (nb0, nb1)
    osums = (osum0, osum1)
    sem_ns = (sem_n0, sem_n1)
    sem_os = (sem_o0, sem_o1)

    shift16 = jnp.full((16,), 16, jnp.int32)
    himask = jnp.full((16,), _HI, jnp.int32)
    rnd = jnp.full((16,), 32768, jnp.int32)

    def lo_f32(v):
        return lax.bitcast_convert_type(lax.shift_left(v, shift16), jnp.float32)

    def hi_f32(v):
        return lax.bitcast_convert_type(lax.bitwise_and(v, himask), jnp.float32)

    def rn_pack(acc_a, acc_b):
        ia = lax.bitcast_convert_type(acc_a, jnp.int32) + rnd
        ib = lax.bitcast_convert_type(acc_b, jnp.int32) + rnd
        return lax.bitwise_or(lax.shift_right_logical(ia, shift16),
                              lax.bitwise_and(ib, himask))

    def accum_block(buf, osum_b):
        @plsc.parallel_loop(0, BLKN)
        def _(r):
            row = r * K
            for g in range(PGRP):
                sl = pl.ds(g * 16, 16)
                acc_a = jnp.zeros((16,), jnp.float32)
                acc_b = jnp.zeros((16,), jnp.float32)
                for j in range(K):
                    v = buf[row + j, sl]
                    acc_a = acc_a + lo_f32(v)
                    acc_b = acc_b + hi_f32(v)
                osum_b[r, sl] = rn_pack(acc_a, acc_b)

    h_t.wait()
    plsc.subcore_barrier()

    for q in range(NRING):
        pltpu.async_copy(tsh.at[neigh_v.at[q]], nbs[q], sem_ns[q])

    svp = (svp0, svp1)
    sem_si = (sem_si0, sem_si1)
    sem_so = (sem_so0, sem_so1)
    h_in = [pltpu.async_copy(tsh.at[nodes_v.at[0]], svp0, sem_si0),
            pltpu.async_copy(tsh.at[nodes_v.at[1]], svp1, sem_si1)]
    for c in range(SCH):
        bsl = c % 2
        h_in[bsl].wait()
        ho = pltpu.async_copy(
            svp[bsl], self_out.at[pl.ds(base + c * SC_C, SC_C)], sem_so[bsl])
        if c + 2 < SCH:
            ho.wait()
            h_in[bsl] = pltpu.async_copy(
                tsh.at[nodes_v.at[c + 2]], svp[bsl], sem_si[bsl])

    def loop_body(i, _):
        for q in range(NRING):
            blk = i * NRING + q
            row0 = base + i * (NRING * BLKN) + q * BLKN
            pltpu.make_async_copy(
                tsh.at[neigh_v.at[blk]], nbs[q], sem_ns[q]).wait()

            @pl.when(i > 0)
            def _():
                pltpu.make_async_copy(
                    osums[q], nsum_out.at[pl.ds(row0 - NRING * BLKN, BLKN)],
                    sem_os[q]).wait()

            accum_block(nbs[q], osums[q])

            @pl.when(i < NIT - 1)
            def _():
                pltpu.async_copy(
                    tsh.at[neigh_v.at[blk + NRING]], nbs[q], sem_ns[q])

            pltpu.async_copy(
                osums[q], nsum_out.at[pl.ds(row0, BLKN)], sem_os[q])
        return 0

    lax.fori_loop(0, NIT, loop_body, 0)

    for q in range(NRING):
        lastq = base + (NIT - 1) * NRING * BLKN + q * BLKN
        pltpu.make_async_copy(
            osums[q], nsum_out.at[pl.ds(lastq, BLKN)], sem_os[q]).wait()
    pltpu.make_async_copy(
        svp0, self_out.at[pl.ds(base + (SCH - 2) * SC_C, SC_C)], sem_so0).wait()
    pltpu.make_async_copy(
        svp1, self_out.at[pl.ds(base + (SCH - 1) * SC_C, SC_C)], sem_so1).wait()


def _mm_body(x1_ref, x2_ref, w1a_ref, w1b_ref, w2a_ref, w2b_ref, b_ref, o_ref):
    x1 = x1_ref[...]
    x2 = x2_ref[...]
    a1 = jax.lax.bitcast_convert_type(
        jax.lax.shift_left(x1, jnp.int32(16)), jnp.float32)
    b1 = jax.lax.bitcast_convert_type(
        jax.lax.bitwise_and(x1, jnp.int32(_HI)), jnp.float32)
    a2 = jax.lax.bitcast_convert_type(
        jax.lax.shift_left(x2, jnp.int32(16)), jnp.float32)
    b2 = jax.lax.bitcast_convert_type(
        jax.lax.bitwise_and(x2, jnp.int32(_HI)), jnp.float32)
    acc = jnp.dot(a1, w1a_ref[...], preferred_element_type=jnp.float32)
    acc = acc + jnp.dot(b1, w1b_ref[...], preferred_element_type=jnp.float32)
    acc = acc + jnp.dot(a2, w2a_ref[...], preferred_element_type=jnp.float32)
    acc = acc + jnp.dot(b2, w2b_ref[...], preferred_element_type=jnp.float32)
    o_ref[...] = jnp.maximum(acc + b_ref[...], 0.0)


_BM = 1000


def _dense(x1, x2, w1a, w1b, w2a, w2b, b2d):
    return pl.pallas_call(
        _mm_body,
        grid=(B // _BM,),
        in_specs=[
            pl.BlockSpec((_BM, DP), lambda i: (i, 0)),
            pl.BlockSpec((_BM, DP), lambda i: (i, 0)),
            pl.BlockSpec((DP, EMB), lambda i: (0, 0)),
            pl.BlockSpec((DP, EMB), lambda i: (0, 0)),
            pl.BlockSpec((DP, EMB), lambda i: (0, 0)),
            pl.BlockSpec((DP, EMB), lambda i: (0, 0)),
            pl.BlockSpec((1, EMB), lambda i: (0, 0)),
        ],
        out_specs=pl.BlockSpec((_BM, EMB), lambda i: (i, 0)),
        out_shape=jax.ShapeDtypeStruct((B, EMB), jnp.float32),
    )(x1, x2, w1a, w1b, w2a, w2b, b2d)


def kernel(table, nodes, neigh_idx, W, b):
    nodes_i = nodes.astype(jnp.int32)
    neigh_i = neigh_idx.astype(jnp.int32)
    pad = BP - B
    nodes_p = jnp.concatenate([nodes_i, jnp.zeros((pad,), jnp.int32)])
    neigh_p = jnp.concatenate([neigh_i, jnp.zeros((pad, K), jnp.int32)])
    nodes_r = nodes_p.reshape(NW, SCH, SC_C)
    neigh_r = neigh_p.reshape(NW, NBLK, BLKR)

    tb = table.astype(jnp.bfloat16)
    tp = tb.reshape(B, D // 32, 2, 16).transpose(0, 1, 3, 2)
    tpack = jax.lax.bitcast_convert_type(tp.reshape(B, DP, 2), jnp.int32)
    tpack = jnp.concatenate(
        [tpack, jnp.zeros((TPAD - B, DP), jnp.int32)])

    selfp, nsump = _sc_gather(nodes_r, neigh_r, tpack)

    g32 = 32 * jnp.arange(D // 32)[:, None]
    p_lo = (g32 + jnp.arange(16)[None, :]).reshape(-1)
    p_hi = (g32 + 16 + jnp.arange(16)[None, :]).reshape(-1)
    w1 = W[:D]
    w2 = W[D:] * (1.0 / K)
    out = _dense(selfp, nsump,
                 w1[p_lo], w1[p_hi], w2[p_lo], w2[p_hi], b.reshape(1, EMB))
    return out
